# baseline (device time: 79198 ns/iter reference)
import jax
import jax.numpy as jnp
from jax import lax
from jax.experimental import pallas as pl
from jax.experimental.pallas import tpu as pltpu

N_DEV = 4
B = 1024
D = 256
CHUNK = B // N_DEV


def kernel(x, Win0, Wout0, Win1, Wout1, Win2, Wout2):
    def body(
        x_ref,
        win0_ref,
        wout0_ref,
        win1_ref,
        wout1_ref,
        win2_ref,
        wout2_ref,
        out_ref,
        xfull,
        pacc,
        rs_send,
        rs_comm,
        ag_send_sems,
        ag_recv_sems,
        rs_send_sems,
        rs_recv_sems,
    ):
        my = lax.axis_index("i")
        left = lax.rem(my + N_DEV - 1, N_DEV)
        right = lax.rem(my + 1, N_DEV)

        barrier_sem = pltpu.get_barrier_semaphore()
        for nbr in (left, right):
            pl.semaphore_signal(
                barrier_sem,
                inc=1,
                device_id=(nbr,),
                device_id_type=pl.DeviceIdType.MESH,
            )
        pl.semaphore_wait(barrier_sem, 2)

        def ring_allgather():
            for h in range(N_DEV - 1):
                o = lax.rem(my - h + N_DEV, N_DEV)
                sl = pl.ds(o * CHUNK, CHUNK)
                rdma = pltpu.make_async_remote_copy(
                    src_ref=xfull.at[sl, :],
                    dst_ref=xfull.at[sl, :],
                    send_sem=ag_send_sems.at[h],
                    recv_sem=ag_recv_sems.at[h],
                    device_id=(right,),
                    device_id_type=pl.DeviceIdType.MESH,
                )
                rdma.start()
                rdma.wait()

        xfull[pl.ds(my * CHUNK, CHUNK), :] = x_ref[...].astype(jnp.bfloat16)
        ring_allgather()

        def layer(win_ref, wout_ref, last):
            w1 = win_ref[...].astype(jnp.bfloat16)
            h = jnp.dot(xfull[...], w1, preferred_element_type=jnp.float32)
            h = jnp.maximum(h, 0.0).astype(jnp.bfloat16)
            w2 = wout_ref[...].astype(jnp.bfloat16)
            pacc[...] = jnp.dot(h, w2, preferred_element_type=jnp.float32)

            for s in range(N_DEV - 1):
                c_send = lax.rem(my - 1 - s + 2 * N_DEV, N_DEV)
                if s == 0:
                    rs_send[...] = pacc[pl.ds(c_send * CHUNK, CHUNK), :].astype(
                        jnp.bfloat16
                    )
                rdma = pltpu.make_async_remote_copy(
                    src_ref=rs_send,
                    dst_ref=rs_comm.at[s],
                    send_sem=rs_send_sems.at[s],
                    recv_sem=rs_recv_sems.at[s],
                    device_id=(right,),
                    device_id_type=pl.DeviceIdType.MESH,
                )
                rdma.start()
                rdma.wait()
                c_recv = lax.rem(my - 2 - s + 2 * N_DEV, N_DEV)
                summed = (
                    pacc[pl.ds(c_recv * CHUNK, CHUNK), :]
                    + rs_comm[s].astype(jnp.float32)
                )
                if s < N_DEV - 2:
                    rs_send[...] = summed.astype(jnp.bfloat16)
                else:
                    xfull[pl.ds(my * CHUNK, CHUNK), :] = summed.astype(
                        jnp.bfloat16
                    )
            ring_allgather()

        layer(win0_ref, wout0_ref, last=False)
        layer(win1_ref, wout1_ref, last=False)
        layer(win2_ref, wout2_ref, last=True)

        out_ref[...] = xfull[...].astype(jnp.float32)

    vmem = pl.BlockSpec(memory_space=pltpu.VMEM)
    return pl.pallas_call(
        body,
        out_shape=jax.ShapeDtypeStruct((B, D), jnp.float32),
        in_specs=[vmem] * 7,
        out_specs=vmem,
        scratch_shapes=[
            pltpu.VMEM((B, D), jnp.bfloat16),
            pltpu.VMEM((B, D), jnp.float32),
            pltpu.VMEM((CHUNK, D), jnp.bfloat16),
            pltpu.VMEM((N_DEV - 1, CHUNK, D), jnp.bfloat16),
            pltpu.SemaphoreType.DMA((N_DEV - 1,)),
            pltpu.SemaphoreType.DMA((N_DEV - 1,)),
            pltpu.SemaphoreType.DMA((N_DEV - 1,)),
            pltpu.SemaphoreType.DMA((N_DEV - 1,)),
        ],
        compiler_params=pltpu.CompilerParams(collective_id=0),
    )(x, Win0, Wout0, Win1, Wout1, Win2, Wout2)


# device time: 47440 ns/iter; 1.6694x vs baseline; 1.6694x over previous
import jax
import jax.numpy as jnp
from jax import lax
from jax.experimental import pallas as pl
from jax.experimental.pallas import tpu as pltpu

N_DEV = 4
B = 1024
D = 256
CHUNK = B // N_DEV


def kernel(x, Win0, Wout0, Win1, Wout1, Win2, Wout2):
    def body(
        x_ref,
        win0_ref,
        wout0_ref,
        win1_ref,
        wout1_ref,
        win2_ref,
        wout2_ref,
        out_ref,
        xfull,
        pacc,
        s_rs,
        r_rs,
        ag_send_sems,
        ag_recv_sems,
        rs_send_sems,
        rs_recv_sems,
    ):
        my = lax.axis_index("i")
        peers = [my ^ 1, 3 - my, my ^ 2]

        barrier_sem = pltpu.get_barrier_semaphore()
        for p in peers:
            pl.semaphore_signal(
                barrier_sem,
                inc=1,
                device_id=(p,),
                device_id_type=pl.DeviceIdType.MESH,
            )
        pl.semaphore_wait(barrier_sem, len(peers))

        wpairs = [
            (win0_ref, wout0_ref),
            (win1_ref, wout1_ref),
            (win2_ref, wout2_ref),
        ]

        def mlp_chunk(w1, w2, start):
            xr = xfull[pl.ds(start, CHUNK), :]
            h = jnp.maximum(
                jnp.dot(xr, w1, preferred_element_type=jnp.float32), 0.0
            )
            pacc[pl.ds(start, CHUNK), :] = jnp.dot(
                h.astype(jnp.bfloat16), w2, preferred_element_type=jnp.float32
            )

        my_sl = pl.ds(my * CHUNK, CHUNK)

        def ag_phase(onchunk):
            descs = []
            for j, p in enumerate(peers):
                d = pltpu.make_async_remote_copy(
                    src_ref=xfull.at[my_sl, :],
                    dst_ref=xfull.at[my_sl, :],
                    send_sem=ag_send_sems.at[j],
                    recv_sem=ag_recv_sems.at[j],
                    device_id=(p,),
                    device_id_type=pl.DeviceIdType.MESH,
                )
                d.start()
                descs.append(d)
            for j, p in enumerate(peers):
                descs[j].wait_recv()
                if onchunk is not None:
                    onchunk(p * CHUNK)
            for d in descs:
                d.wait_send()

        xfull[my_sl, :] = x_ref[...].astype(jnp.bfloat16)
        w1 = wpairs[0][0][...].astype(jnp.bfloat16)
        w2 = wpairs[0][1][...].astype(jnp.bfloat16)
        ag_phase(lambda s, a=w1, b=w2: mlp_chunk(a, b, s))

        for l in range(3):
            rs_descs = []
            for j, p in enumerate(peers):
                s_rs[j, :, :] = pacc[pl.ds(p * CHUNK, CHUNK), :].astype(
                    jnp.bfloat16
                )
                d = pltpu.make_async_remote_copy(
                    src_ref=s_rs.at[j],
                    dst_ref=r_rs.at[j],
                    send_sem=rs_send_sems.at[j],
                    recv_sem=rs_recv_sems.at[j],
                    device_id=(p,),
                    device_id_type=pl.DeviceIdType.MESH,
                )
                d.start()
                rs_descs.append(d)
            mlp_chunk(w1, w2, my * CHUNK)
            for d in rs_descs:
                d.wait_recv()
            own = pacc[my_sl, :]
            for j in range(3):
                own = own + r_rs[j].astype(jnp.float32)
            xfull[my_sl, :] = own.astype(jnp.bfloat16)
            for d in rs_descs:
                d.wait_send()

            if l < 2:
                w1 = wpairs[l + 1][0][...].astype(jnp.bfloat16)
                w2 = wpairs[l + 1][1][...].astype(jnp.bfloat16)
                ag_phase(lambda s, a=w1, b=w2: mlp_chunk(a, b, s))
            else:
                ag_phase(None)

        out_ref[...] = xfull[...].astype(jnp.float32)

    vmem = pl.BlockSpec(memory_space=pltpu.VMEM)
    return pl.pallas_call(
        body,
        out_shape=jax.ShapeDtypeStruct((B, D), jnp.float32),
        in_specs=[vmem] * 7,
        out_specs=vmem,
        scratch_shapes=[
            pltpu.VMEM((B, D), jnp.bfloat16),
            pltpu.VMEM((B, D), jnp.float32),
            pltpu.VMEM((3, CHUNK, D), jnp.bfloat16),
            pltpu.VMEM((3, CHUNK, D), jnp.bfloat16),
            pltpu.SemaphoreType.DMA((3,)),
            pltpu.SemaphoreType.DMA((3,)),
            pltpu.SemaphoreType.DMA((3,)),
            pltpu.SemaphoreType.DMA((3,)),
        ],
        compiler_params=pltpu.CompilerParams(collective_id=0),
    )(x, Win0, Wout0, Win1, Wout1, Win2, Wout2)


# device time: 44835 ns/iter; 1.7664x vs baseline; 1.0581x over previous
import jax
import jax.numpy as jnp
from jax import lax
from jax.experimental import pallas as pl
from jax.experimental.pallas import tpu as pltpu

N_DEV = 4
B = 1024
D = 256
CHUNK = B // N_DEV


def kernel(x, Win0, Wout0, Win1, Wout1, Win2, Wout2):
    def body(
        x_ref,
        win0_ref,
        wout0_ref,
        win1_ref,
        wout1_ref,
        win2_ref,
        wout2_ref,
        out_ref,
        xfull,
        pacc,
        s_rs,
        r_rs,
        ag_send_sems,
        ag_recv_sems,
        rs_send_sems,
        rs_recv_sems,
    ):
        my = lax.axis_index("i")
        peers = [my ^ 1, 3 - my, my ^ 2]

        barrier_sem = pltpu.get_barrier_semaphore()
        for p in peers:
            pl.semaphore_signal(
                barrier_sem,
                inc=1,
                device_id=(p,),
                device_id_type=pl.DeviceIdType.MESH,
            )
        pl.semaphore_wait(barrier_sem, len(peers))

        wpairs = [
            (win0_ref, wout0_ref),
            (win1_ref, wout1_ref),
            (win2_ref, wout2_ref),
        ]

        def mlp_chunk(w1, w2, start):
            xr = xfull[pl.ds(start, CHUNK), :]
            h = jnp.maximum(
                jnp.dot(xr, w1, preferred_element_type=jnp.float32), 0.0
            )
            pacc[pl.ds(start, CHUNK), :] = jnp.dot(
                h.astype(jnp.bfloat16), w2, preferred_element_type=jnp.float32
            )

        my_sl = pl.ds(my * CHUNK, CHUNK)

        def start_ag():
            descs = []
            for j, p in enumerate(peers):
                d = pltpu.make_async_remote_copy(
                    src_ref=xfull.at[my_sl, :],
                    dst_ref=xfull.at[my_sl, :],
                    send_sem=ag_send_sems.at[j],
                    recv_sem=ag_recv_sems.at[j],
                    device_id=(p,),
                    device_id_type=pl.DeviceIdType.MESH,
                )
                d.start()
                descs.append(d)
            return descs

        def start_rs(j, p):
            s_rs[j, :, :] = pacc[pl.ds(p * CHUNK, CHUNK), :].astype(
                jnp.bfloat16
            )
            d = pltpu.make_async_remote_copy(
                src_ref=s_rs.at[j],
                dst_ref=r_rs.at[j],
                send_sem=rs_send_sems.at[j],
                recv_sem=rs_recv_sems.at[j],
                device_id=(p,),
                device_id_type=pl.DeviceIdType.MESH,
            )
            d.start()
            return d

        xfull[my_sl, :] = x_ref[...].astype(jnp.bfloat16)
        w1 = wpairs[0][0][...].astype(jnp.bfloat16)
        w2 = wpairs[0][1][...].astype(jnp.bfloat16)
        ag_descs = start_ag()
        rs_descs = [None, None, None]
        for j, p in enumerate(peers):
            ag_descs[j].wait_recv()
            mlp_chunk(w1, w2, p * CHUNK)
            rs_descs[j] = start_rs(j, p)
        for d in ag_descs:
            d.wait_send()

        for l in range(3):
            mlp_chunk(w1, w2, my * CHUNK)
            for d in rs_descs:
                d.wait_recv()
            for d in rs_descs:
                d.wait_send()
            own = pacc[my_sl, :]
            for j in range(3):
                own = own + r_rs[j].astype(jnp.float32)
            xfull[my_sl, :] = own.astype(jnp.bfloat16)
            if l == 2:
                out_ref[my_sl, :] = own

            ag_descs = start_ag()
            if l < 2:
                w1 = wpairs[l + 1][0][...].astype(jnp.bfloat16)
                w2 = wpairs[l + 1][1][...].astype(jnp.bfloat16)
                for j, p in enumerate(peers):
                    ag_descs[j].wait_recv()
                    mlp_chunk(w1, w2, p * CHUNK)
                    rs_descs[j] = start_rs(j, p)
            else:
                for j, p in enumerate(peers):
                    ag_descs[j].wait_recv()
                    p_sl = pl.ds(p * CHUNK, CHUNK)
                    out_ref[p_sl, :] = xfull[p_sl, :].astype(jnp.float32)
            for d in ag_descs:
                d.wait_send()

    vmem = pl.BlockSpec(memory_space=pltpu.VMEM)
    return pl.pallas_call(
        body,
        out_shape=jax.ShapeDtypeStruct((B, D), jnp.float32),
        in_specs=[vmem] * 7,
        out_specs=vmem,
        scratch_shapes=[
            pltpu.VMEM((B, D), jnp.bfloat16),
            pltpu.VMEM((B, D), jnp.float32),
            pltpu.VMEM((3, CHUNK, D), jnp.bfloat16),
            pltpu.VMEM((3, CHUNK, D), jnp.bfloat16),
            pltpu.SemaphoreType.DMA((3,)),
            pltpu.SemaphoreType.DMA((3,)),
            pltpu.SemaphoreType.DMA((3,)),
            pltpu.SemaphoreType.DMA((3,)),
        ],
        compiler_params=pltpu.CompilerParams(collective_id=0),
    )(x, Win0, Wout0, Win1, Wout1, Win2, Wout2)


# device time: 37004 ns/iter; 2.1403x vs baseline; 1.2116x over previous
import jax
import jax.numpy as jnp
from jax import lax
from jax.experimental import pallas as pl
from jax.experimental.pallas import tpu as pltpu

N_DEV = 4
B = 1024
D = 256
CHUNK = B // N_DEV
NH = 4
PIECE = CHUNK // NH


def kernel(x, Win0, Wout0, Win1, Wout1, Win2, Wout2):
    x, Win0, Wout0, Win1, Wout1, Win2, Wout2 = (
        a.astype(jnp.bfloat16)
        for a in (x, Win0, Wout0, Win1, Wout1, Win2, Wout2)
    )

    def body(
        x_ref,
        win0_ref,
        wout0_ref,
        win1_ref,
        wout1_ref,
        win2_ref,
        wout2_ref,
        out_ref,
        xfull,
        pacc,
        r_rs,
        ag_send_sems,
        ag_recv_sems,
        rs_send_sems,
        rs_recv_sems,
    ):
        my = lax.axis_index("i")
        peers = [my ^ 2, my ^ 1, 3 - my]
        wait_order = [1, 2, 0]

        barrier_sem = pltpu.get_barrier_semaphore()
        for p in peers:
            pl.semaphore_signal(
                barrier_sem,
                inc=1,
                device_id=(p,),
                device_id_type=pl.DeviceIdType.MESH,
            )
        pl.semaphore_wait(barrier_sem, len(peers))

        wpairs = [
            (win0_ref, wout0_ref),
            (win1_ref, wout1_ref),
            (win2_ref, wout2_ref),
        ]

        def mlp_rows(w1, w2, start, n):
            xr = xfull[pl.ds(start, n), :]
            h = jnp.maximum(
                jnp.dot(xr, w1, preferred_element_type=jnp.float32), 0.0
            ).astype(jnp.bfloat16)
            pacc[pl.ds(start, n), :] = jnp.dot(
                h, w2, preferred_element_type=jnp.float32
            ).astype(jnp.bfloat16)

        my_sl = pl.ds(my * CHUNK, CHUNK)

        def start_ag_piece(buf, h):
            descs = []
            for j, p in enumerate(peers):
                sl = pl.ds(my * CHUNK + h * PIECE, PIECE)
                d = pltpu.make_async_remote_copy(
                    src_ref=buf.at[sl, :],
                    dst_ref=buf.at[sl, :],
                    send_sem=ag_send_sems.at[j * NH + h],
                    recv_sem=ag_recv_sems.at[j * NH + h],
                    device_id=(p,),
                    device_id_type=pl.DeviceIdType.MESH,
                )
                d.start()
                descs.append(d)
            return descs

        def start_rs_piece(j, p, h):
            d = pltpu.make_async_remote_copy(
                src_ref=pacc.at[pl.ds(p * CHUNK + h * PIECE, PIECE), :],
                dst_ref=r_rs.at[j, pl.ds(h * PIECE, PIECE), :],
                send_sem=rs_send_sems.at[j * NH + h],
                recv_sem=rs_recv_sems.at[j * NH + h],
                device_id=(p,),
                device_id_type=pl.DeviceIdType.MESH,
            )
            d.start()
            return d

        rs_descs = [None] * (3 * NH)

        def ag_arrivals_compute(ag_descs, w1, w2):
            for h in range(NH):
                for j in wait_order:
                    p = peers[j]
                    ag_descs[h][j].wait_recv()
                    mlp_rows(w1, w2, p * CHUNK + h * PIECE, PIECE)
                    rs_descs[j * NH + h] = start_rs_piece(j, p, h)

        xfull[my_sl, :] = x_ref[...]
        ag_descs = [start_ag_piece(xfull, h) for h in range(NH)]
        w1 = wpairs[0][0][...]
        w2 = wpairs[0][1][...]
        ag_arrivals_compute(ag_descs, w1, w2)
        for hd in ag_descs:
            for d in hd:
                d.wait_send()

        for l in range(3):
            dst = xfull if l < 2 else out_ref
            new_ag = []
            for h in range(NH):
                mlp_rows(w1, w2, my * CHUNK + h * PIECE, PIECE)
                for j in range(3):
                    rs_descs[j * NH + h].wait_recv()
                h_sl = pl.ds(my * CHUNK + h * PIECE, PIECE)
                own = pacc[h_sl, :].astype(jnp.float32)
                for j in range(3):
                    own = own + r_rs[j, pl.ds(h * PIECE, PIECE), :].astype(
                        jnp.float32
                    )
                dst[h_sl, :] = own.astype(jnp.bfloat16)
                new_ag.append(start_ag_piece(dst, h))
            ag_descs = new_ag
            for d in rs_descs:
                d.wait_send()

            if l < 2:
                w1 = wpairs[l + 1][0][...]
                w2 = wpairs[l + 1][1][...]
                ag_arrivals_compute(ag_descs, w1, w2)
            else:
                for hd in ag_descs:
                    for d in hd:
                        d.wait_recv()
            for hd in ag_descs:
                for d in hd:
                    d.wait_send()

    vmem = pl.BlockSpec(memory_space=pltpu.VMEM)
    return pl.pallas_call(
        body,
        out_shape=jax.ShapeDtypeStruct((B, D), jnp.bfloat16),
        in_specs=[vmem] * 7,
        out_specs=vmem,
        scratch_shapes=[
            pltpu.VMEM((B, D), jnp.bfloat16),
            pltpu.VMEM((B, D), jnp.bfloat16),
            pltpu.VMEM((3, CHUNK, D), jnp.bfloat16),
            pltpu.SemaphoreType.DMA((3 * NH,)),
            pltpu.SemaphoreType.DMA((3 * NH,)),
            pltpu.SemaphoreType.DMA((3 * NH,)),
            pltpu.SemaphoreType.DMA((3 * NH,)),
        ],
        compiler_params=pltpu.CompilerParams(collective_id=0),
    )(x, Win0, Wout0, Win1, Wout1, Win2, Wout2)
